# Initial kernel scaffold; baseline (speedup 1.0000x reference)
import jax
import jax.numpy as jnp
from jax.experimental import pallas as pl


def kernel(x, v_first, capital_shares, step, warmup_steps, ln1_g, ln1_b, ln2_g, ln2_b, Wr, Wk, Wv, Wo, Ws, conf_W, W1, W2, Wmix, Wd, Wa, Wb1, Wb2):
    raise NotImplementedError("write your pallas kernel here")


# speedup vs baseline: 1.2256x; 1.2256x over previous
import functools

import jax
import jax.numpy as jnp
from jax import lax
from jax.experimental import pallas as pl
from jax.experimental.pallas import tpu as pltpu

_B, _T, _C = 2, 2048, 768
_NUM_RWKV, _NUM_TRANS = 6, 2
_E = _NUM_RWKV + _NUM_TRANS
_N = _B * _T

_BLK1 = 512
_NB1 = _N // _BLK1

_F32 = jnp.float32
_BF16 = jnp.bfloat16


def _bdot(a, b):
    return lax.dot_general(a, b, (((1,), (0,)), ((), ())),
                           preferred_element_type=_F32)


def _ln_rows(x, g, b):
    mu = jnp.mean(x, axis=1, keepdims=True)
    d = x - mu
    var = jnp.mean(d * d, axis=1, keepdims=True)
    return d / jnp.sqrt(var + 1e-5) * g + b


def _k1_body(x_ref, vf_ref, wr_ref, wk_ref, wv_ref, wo_ref, ws_ref,
             cw_ref, wa_ref, wd_ref, wb1_ref, wb2_ref,
             l1g_ref, l1b_ref, l2g_ref, l2b_ref, lcap_ref,
             x1_ref, hs_ref, ss_ref, win_ref, cost_ref, sse_ref):
    i = pl.program_id(0)
    xf = x_ref[...]
    xn = _ln_rows(xf, l1g_ref[...], l1b_ref[...])
    xb = xn.astype(_BF16)
    r = _bdot(xb, wr_ref[...])
    k = _bdot(xb, wk_ref[...])
    v0 = _bdot(xb, wv_ref[...])
    v = v0 + (vf_ref[...] - v0) * jax.nn.sigmoid(k)
    att = _bdot((jax.nn.sigmoid(r) * v).astype(_BF16), wo_ref[...])
    state = jnp.tanh(_bdot(xb, ws_ref[...]))
    x1 = xf + att
    x1_ref[...] = x1
    h = _ln_rows(x1, l2g_ref[...], l2b_ref[...])
    hb = h.astype(_BF16)
    conf = jax.nn.sigmoid(_bdot(hb, cw_ref[...]))
    aff = _bdot(hb, wa_ref[...])
    dcol = _bdot(hb, wd_ref[...])[:, 0:1]
    diff = jax.nn.softplus(dcol)
    eff = conf * diff + 0.1 * aff + lcap_ref[...]
    costs = jnp.max(eff, axis=1)
    ids = lax.broadcasted_iota(jnp.int32, (_BLK1, _E), 1)
    win = jnp.min(jnp.where(eff == costs[:, None], ids, _E), axis=1)
    win_ref[...] = win[:, None]
    cost_ref[...] = costs[:, None]
    wc = jnp.sum(jnp.where(ids == win[:, None], conf, 0.0), axis=1)
    scale = (wc / (wc + 1e-6))[:, None]
    hs_ref[...] = (h * scale).astype(_BF16)
    ss_ref[...] = (state * scale).astype(_BF16)
    t1 = jnp.tanh(_bdot(state.astype(_BF16), wb1_ref[...]))
    recon = _bdot(t1.astype(_BF16), wb2_ref[...])
    dsse = jnp.sum((recon - h) ** 2)

    @pl.when(i == 0)
    def _():
        sse_ref[0, 0] = dsse

    @pl.when(i != 0)
    def _():
        sse_ref[0, 0] += dsse


def _k1(x2d, vf2d, Wr, Wk, Wv, Wo, Ws, cwT, Wa, Wd8, Wb1, Wb2,
        l1g, l1b, l2g, l2b, lcap):
    row_spec = pl.BlockSpec((_BLK1, _C), lambda i: (i, 0))
    full = lambda a: pl.BlockSpec(a.shape, lambda i: tuple(0 for _ in a.shape))
    out_shapes = (
        jax.ShapeDtypeStruct((_N, _C), _F32),
        jax.ShapeDtypeStruct((_N, _C), _BF16),
        jax.ShapeDtypeStruct((_N, _C), _BF16),
        jax.ShapeDtypeStruct((_N, 1), jnp.int32),
        jax.ShapeDtypeStruct((_N, 1), _F32),
        jax.ShapeDtypeStruct((1, 1), _F32),
    )
    out_specs = (
        row_spec,
        row_spec,
        row_spec,
        pl.BlockSpec((_BLK1, 1), lambda i: (i, 0)),
        pl.BlockSpec((_BLK1, 1), lambda i: (i, 0)),
        pl.BlockSpec((1, 1), lambda i: (0, 0), memory_space=pltpu.SMEM),
    )
    ws = [Wr, Wk, Wv, Wo, Ws, cwT, Wa, Wd8, Wb1, Wb2, l1g, l1b, l2g, l2b, lcap]
    return pl.pallas_call(
        _k1_body,
        grid=(_NB1,),
        in_specs=[row_spec, row_spec] + [full(w) for w in ws],
        out_specs=out_specs,
        out_shape=out_shapes,
        compiler_params=pltpu.CompilerParams(
            dimension_semantics=("arbitrary",)),
    )(x2d, vf2d, *ws)


def kernel(x, v_first, capital_shares, step, warmup_steps, ln1_g, ln1_b, ln2_g, ln2_b, Wr, Wk, Wv, Wo, Ws, conf_W, W1, W2, Wmix, Wd, Wa, Wb1, Wb2):
    C = _C
    x2d = x.reshape(_N, C)
    vf2d = v_first.reshape(_N, C)
    bf = lambda w: w.astype(_BF16)
    Wd8 = jnp.pad(Wd, ((0, 0), (0, 7)))
    lcap = jnp.log(capital_shares + 1e-6)[None, :]
    x1, hs, ss, win2d, cost2d, sse = _k1(
        x2d, vf2d, bf(Wr), bf(Wk), bf(Wv), bf(Wo), bf(Ws),
        bf(conf_W.T), bf(Wa), bf(Wd8), bf(Wb1), bf(Wb2),
        ln1_g[None, :], ln1_b[None, :], ln2_g[None, :], ln2_b[None, :], lcap)
    winners = win2d[:, 0]
    costs = cost2d[:, 0]
    recon_loss = sse[0, 0] / (_N * C)

    W1b, W2b, Wmb = bf(W1), bf(W2), bf(Wmix)
    final = jnp.zeros((_N, C), _F32)
    for e in range(_E):
        mask = winners == e
        if e >= _NUM_RWKV:
            hh_s = hs.astype(_F32) + _bdot(ss, Wmb[e - _NUM_RWKV])
        else:
            hh_s = hs.astype(_F32)
        u = jax.nn.relu(_bdot(hh_s.astype(_BF16), W1b[e]))
        eo_s = _bdot(u.astype(_BF16), W2b[e])
        final = jnp.where(mask[:, None], eo_s, final)
    out = (x1 + final).reshape(_B, _T, C)
    return (out, v_first, winners.reshape(_B, _T), costs.reshape(_B, _T),
            recon_loss)


# speedup vs baseline: 1.2771x; 1.0420x over previous
import functools

import jax
import jax.numpy as jnp
from jax import lax
from jax.experimental import pallas as pl
from jax.experimental.pallas import tpu as pltpu

_B, _T, _C = 2, 2048, 768
_NUM_RWKV, _NUM_TRANS = 6, 2
_E = _NUM_RWKV + _NUM_TRANS
_N = _B * _T

_BLK1 = 512
_NB1 = _N // _BLK1

_F32 = jnp.float32
_BF16 = jnp.bfloat16


def _bdot(a, b):
    return lax.dot_general(a, b, (((1,), (0,)), ((), ())),
                           preferred_element_type=_F32)


def _ln_rows(x, g, b):
    mu = jnp.mean(x, axis=1, keepdims=True)
    d = x - mu
    var = jnp.mean(d * d, axis=1, keepdims=True)
    return d / jnp.sqrt(var + 1e-5) * g + b


def _k1_body(x_ref, vf_ref, wr_ref, wk_ref, wv_ref, wo_ref, ws_ref,
             cw_ref, wa_ref, wd_ref, wb1_ref, wb2_ref,
             l1g_ref, l1b_ref, l2g_ref, l2b_ref, lcap_ref,
             x1_ref, hs_ref, ss_ref, win_ref, cost_ref, sse_ref):
    i = pl.program_id(0)
    xf = x_ref[...]
    xn = _ln_rows(xf, l1g_ref[...], l1b_ref[...])
    xb = xn.astype(_BF16)
    r = _bdot(xb, wr_ref[...])
    k = _bdot(xb, wk_ref[...])
    v0 = _bdot(xb, wv_ref[...])
    v = v0 + (vf_ref[...] - v0) * jax.nn.sigmoid(k)
    att = _bdot((jax.nn.sigmoid(r) * v).astype(_BF16), wo_ref[...])
    state = jnp.tanh(_bdot(xb, ws_ref[...]))
    x1 = xf + att
    x1_ref[...] = x1
    h = _ln_rows(x1, l2g_ref[...], l2b_ref[...])
    hb = h.astype(_BF16)
    conf = jax.nn.sigmoid(_bdot(hb, cw_ref[...]))
    aff = _bdot(hb, wa_ref[...])
    dcol = _bdot(hb, wd_ref[...])[:, 0:1]
    diff = jax.nn.softplus(dcol)
    eff = conf * diff + 0.1 * aff + lcap_ref[...]
    costs = jnp.max(eff, axis=1)
    ids = lax.broadcasted_iota(jnp.int32, (_BLK1, _E), 1)
    win = jnp.min(jnp.where(eff == costs[:, None], ids, _E), axis=1)
    win_ref[...] = win[:, None]
    cost_ref[...] = costs[:, None]
    wc = jnp.sum(jnp.where(ids == win[:, None], conf, 0.0), axis=1)
    scale = (wc / (wc + 1e-6))[:, None]
    hs_ref[...] = (h * scale).astype(_BF16)
    ss_ref[...] = (state * scale).astype(_BF16)
    t1 = jnp.tanh(_bdot(state.astype(_BF16), wb1_ref[...]))
    recon = _bdot(t1.astype(_BF16), wb2_ref[...])
    dsse = jnp.sum((recon - h) ** 2)

    @pl.when(i == 0)
    def _():
        sse_ref[0, 0] = dsse

    @pl.when(i != 0)
    def _():
        sse_ref[0, 0] += dsse


def _k1(x2d, vf2d, Wr, Wk, Wv, Wo, Ws, cwT, Wa, Wd8, Wb1, Wb2,
        l1g, l1b, l2g, l2b, lcap):
    row_spec = pl.BlockSpec((_BLK1, _C), lambda i: (i, 0))
    full = lambda a: pl.BlockSpec(a.shape, lambda i: tuple(0 for _ in a.shape))
    out_shapes = (
        jax.ShapeDtypeStruct((_N, _C), _F32),
        jax.ShapeDtypeStruct((_N, _C), _BF16),
        jax.ShapeDtypeStruct((_N, _C), _BF16),
        jax.ShapeDtypeStruct((_N, 1), jnp.int32),
        jax.ShapeDtypeStruct((_N, 1), _F32),
        jax.ShapeDtypeStruct((1, 1), _F32),
    )
    out_specs = (
        row_spec,
        row_spec,
        row_spec,
        pl.BlockSpec((_BLK1, 1), lambda i: (i, 0)),
        pl.BlockSpec((_BLK1, 1), lambda i: (i, 0)),
        pl.BlockSpec((1, 1), lambda i: (0, 0), memory_space=pltpu.SMEM),
    )
    ws = [Wr, Wk, Wv, Wo, Ws, cwT, Wa, Wd8, Wb1, Wb2, l1g, l1b, l2g, l2b, lcap]
    return pl.pallas_call(
        _k1_body,
        grid=(_NB1,),
        in_specs=[row_spec, row_spec] + [full(w) for w in ws],
        out_specs=out_specs,
        out_shape=out_shapes,
        compiler_params=pltpu.CompilerParams(
            dimension_semantics=("arbitrary",)),
    )(x2d, vf2d, *ws)


_R2, _C2 = 32, 128


def _k2_body(w_ref, pos_ref, offs_ref):
    w = w_ref[...]
    iu = lax.broadcasted_iota(jnp.int32, (_C2, _C2), 0)
    ju = lax.broadcasted_iota(jnp.int32, (_C2, _C2), 1)
    U = (iu < ju).astype(_BF16)
    il = lax.broadcasted_iota(jnp.int32, (_R2, _R2), 0)
    jl = lax.broadcasted_iota(jnp.int32, (_R2, _R2), 1)
    L = (jl < il).astype(_BF16)
    pos = jnp.zeros((_R2, _C2), _F32)
    off = jnp.float32(0.0)
    for e in range(_E):
        m = (w == e).astype(_F32)
        pref = _bdot(m.astype(_BF16), U)
        tot = jnp.sum(m, axis=1, keepdims=True)
        rowpref = _bdot(L, tot.astype(_BF16))
        offs_ref[e, 0] = off.astype(jnp.int32)
        pos = pos + m * (off + rowpref + pref)
        off = off + jnp.sum(m)
    for e in range(_E, 16):
        offs_ref[e, 0] = jnp.int32(_N)
    pos_ref[...] = pos.astype(jnp.int32)


def _k2(win2d32):
    return pl.pallas_call(
        _k2_body,
        grid=(1,),
        in_specs=[pl.BlockSpec((_R2, _C2), lambda i: (0, 0))],
        out_specs=(
            pl.BlockSpec((_R2, _C2), lambda i: (0, 0)),
            pl.BlockSpec((16, 1), lambda i: (0, 0), memory_space=pltpu.SMEM),
        ),
        out_shape=(
            jax.ShapeDtypeStruct((_R2, _C2), jnp.int32),
            jax.ShapeDtypeStruct((16, 1), jnp.int32),
        ),
    )(win2d32)


_M4 = 256
_NB4 = _N // _M4
_G4 = _NB4 + _E - 1


def _k4_body(bm_r, exc_r, wmx_r, first_r, exraw_r, offs_r,
             hs_r, ss_r, x1_r, w1_r, w2_r, wm_r, out_r, acc_r):
    g = pl.program_id(0)
    e = exraw_r[g]
    lo = offs_r[e]
    hi = offs_r[e + 1]
    istrans = e >= _NUM_RWKV

    @pl.when(istrans)
    def _():
        acc_r[...] = hs_r[...].astype(_F32) + _bdot(ss_r[...], wm_r[0])

    @pl.when(jnp.logical_not(istrans))
    def _():
        acc_r[...] = hs_r[...].astype(_F32)

    u = jax.nn.relu(_bdot(acc_r[...].astype(_BF16), w1_r[0]))
    eo = _bdot(u.astype(_BF16), w2_r[0])
    rows = bm_r[g] * _M4 + lax.broadcasted_iota(jnp.int32, (_M4, 1), 0)
    inb = jnp.logical_and(rows >= lo, rows < hi)
    contrib = jnp.where(inb, eo, 0.0)
    isfirst = first_r[g] == 1

    @pl.when(isfirst)
    def _():
        out_r[...] = x1_r[...] + contrib

    @pl.when(jnp.logical_not(isfirst))
    def _():
        out_r[...] += contrib


def _k4(bm, exc, wmx, first, exraw, offs, hs_s, ss_s, x1_s, W1b, W2b, Wmb):
    row_spec = pl.BlockSpec((_M4, _C), lambda g, *s: (s[0][g], 0))
    grid_spec = pltpu.PrefetchScalarGridSpec(
        num_scalar_prefetch=6,
        grid=(_G4,),
        in_specs=[
            row_spec, row_spec, row_spec,
            pl.BlockSpec((1, _C, _C), lambda g, *s: (s[1][g], 0, 0)),
            pl.BlockSpec((1, _C, _C), lambda g, *s: (s[1][g], 0, 0)),
            pl.BlockSpec((1, _C, _C), lambda g, *s: (s[2][g], 0, 0)),
        ],
        out_specs=pl.BlockSpec((_M4, _C), lambda g, *s: (s[0][g], 0)),
        scratch_shapes=[pltpu.VMEM((_M4, _C), _F32)],
    )
    return pl.pallas_call(
        _k4_body,
        grid_spec=grid_spec,
        out_shape=jax.ShapeDtypeStruct((_N, _C), _F32),
        compiler_params=pltpu.CompilerParams(
            dimension_semantics=("arbitrary",)),
    )(bm, exc, wmx, first, exraw, offs, hs_s, ss_s, x1_s, W1b, W2b, Wmb)


def _schedule(offs):
    starts = offs[0:_E]
    ends = offs[1:_E + 1]
    b = jnp.arange(_NB4, dtype=jnp.int32)[:, None]
    inc = jnp.logical_and(starts[None, :] < (b + 1) * _M4,
                          ends[None, :] > b * _M4)
    tot = jnp.cumsum(inc.reshape(-1).astype(jnp.int32))
    kg = jnp.searchsorted(tot, jnp.arange(1, _G4 + 1, dtype=jnp.int32),
                          side='left').astype(jnp.int32)
    valid = jnp.arange(_G4, dtype=jnp.int32) < tot[-1]
    bm = jnp.where(valid, kg // _E, _NB4 - 1).astype(jnp.int32)
    ex = jnp.where(valid, kg % _E, _E).astype(jnp.int32)
    exc = jnp.minimum(ex, _E - 1)
    wmx = jnp.clip(ex - _NUM_RWKV, 0, 1)
    first = jnp.concatenate(
        [jnp.ones((1,), jnp.int32), (bm[1:] != bm[:-1]).astype(jnp.int32)])
    return bm, exc, wmx, first, ex


def kernel(x, v_first, capital_shares, step, warmup_steps, ln1_g, ln1_b, ln2_g, ln2_b, Wr, Wk, Wv, Wo, Ws, conf_W, W1, W2, Wmix, Wd, Wa, Wb1, Wb2):
    C = _C
    x2d = x.reshape(_N, C)
    vf2d = v_first.reshape(_N, C)
    bf = lambda w: w.astype(_BF16)
    Wd8 = jnp.pad(Wd, ((0, 0), (0, 7)))
    lcap = jnp.log(capital_shares + 1e-6)[None, :]
    x1, hs, ss, win2d, cost2d, sse = _k1(
        x2d, vf2d, bf(Wr), bf(Wk), bf(Wv), bf(Wo), bf(Ws),
        bf(conf_W.T), bf(Wa), bf(Wd8), bf(Wb1), bf(Wb2),
        ln1_g[None, :], ln1_b[None, :], ln2_g[None, :], ln2_b[None, :], lcap)
    winners = win2d[:, 0]
    costs = cost2d[:, 0]
    recon_loss = sse[0, 0] / (_N * C)

    pos2d, offs16 = _k2(win2d.reshape(_R2, _C2))
    pos = pos2d.reshape(_N)
    offs = offs16[:, 0]
    bm, exc, wmx, first, exraw = _schedule(offs)
    order = jnp.zeros((_N,), jnp.int32).at[pos].set(
        jnp.arange(_N, dtype=jnp.int32))
    hs_s = jnp.take(hs, order, axis=0)
    ss_s = jnp.take(ss, order, axis=0)
    x1_s = jnp.take(x1, order, axis=0)
    out_sorted = _k4(bm, exc, wmx, first, exraw, offs,
                     hs_s, ss_s, x1_s, bf(W1), bf(W2), bf(Wmix))
    out = jnp.take(out_sorted, pos, axis=0).reshape(_B, _T, C)
    return (out, v_first, winners.reshape(_B, _T), costs.reshape(_B, _T),
            recon_loss)


# speedup vs baseline: 1.3683x; 1.0714x over previous
import functools

import jax
import jax.numpy as jnp
from jax import lax
from jax.experimental import pallas as pl
from jax.experimental.pallas import tpu as pltpu

_B, _T, _C = 2, 2048, 768
_NUM_RWKV, _NUM_TRANS = 6, 2
_E = _NUM_RWKV + _NUM_TRANS
_N = _B * _T

_BLK1 = 512
_NB1 = _N // _BLK1

_F32 = jnp.float32
_BF16 = jnp.bfloat16


def _bdot(a, b):
    return lax.dot_general(a, b, (((1,), (0,)), ((), ())),
                           preferred_element_type=_F32)


def _ddot(a, b):
    return lax.dot_general(a, b, (((1,), (0,)), ((), ())),
                           precision=lax.Precision.DEFAULT,
                           preferred_element_type=_F32)


def _ln_rows(x, g, b):
    mu = jnp.mean(x, axis=1, keepdims=True)
    d = x - mu
    var = jnp.mean(d * d, axis=1, keepdims=True)
    return d / jnp.sqrt(var + 1e-5) * g + b


def _k1_body(x_ref, vf_ref, wr_ref, wk_ref, wv_ref, wo_ref, ws_ref,
             cw_ref, wa_ref, wd_ref, wb1_ref, wb2_ref,
             l1g_ref, l1b_ref, l2g_ref, l2b_ref, lcap_ref,
             x1_ref, hs_ref, ss_ref, win_ref, cost_ref, sse_ref):
    i = pl.program_id(0)
    xf = x_ref[...]
    xn = _ln_rows(xf, l1g_ref[...], l1b_ref[...])
    r = _ddot(xn, wr_ref[...])
    k = _ddot(xn, wk_ref[...])
    v0 = _ddot(xn, wv_ref[...])
    v = v0 + (vf_ref[...] - v0) * jax.nn.sigmoid(k)
    att = _ddot(jax.nn.sigmoid(r) * v, wo_ref[...])
    state = jnp.tanh(_ddot(xn, ws_ref[...]))
    x1 = xf + att
    x1_ref[...] = x1
    h = _ln_rows(x1, l2g_ref[...], l2b_ref[...])
    conf = jax.nn.sigmoid(_ddot(h, cw_ref[...]))
    aff = _ddot(h, wa_ref[...])
    dcol = _ddot(h, wd_ref[...])[:, 0:1]
    diff = jax.nn.softplus(dcol)
    eff = conf * diff + 0.1 * aff + lcap_ref[...]
    costs = jnp.max(eff, axis=1)
    ids = lax.broadcasted_iota(jnp.int32, (_BLK1, _E), 1)
    win = jnp.min(jnp.where(eff == costs[:, None], ids, _E), axis=1)
    win_ref[...] = win[:, None]
    cost_ref[...] = costs[:, None]
    wc = jnp.sum(jnp.where(ids == win[:, None], conf, 0.0), axis=1)
    scale = (wc / (wc + 1e-6))[:, None]
    hs_ref[...] = (h * scale).astype(_BF16)
    ss_ref[...] = (state * scale).astype(_BF16)
    t1 = jnp.tanh(_ddot(state, wb1_ref[...]))
    recon = _ddot(t1, wb2_ref[...])
    dsse = jnp.sum((recon - h) ** 2)

    @pl.when(i == 0)
    def _():
        sse_ref[0, 0] = dsse

    @pl.when(i != 0)
    def _():
        sse_ref[0, 0] += dsse


def _k1(x2d, vf2d, Wr, Wk, Wv, Wo, Ws, cwT, Wa, Wd8, Wb1, Wb2,
        l1g, l1b, l2g, l2b, lcap):
    row_spec = pl.BlockSpec((_BLK1, _C), lambda i: (i, 0))
    full = lambda a: pl.BlockSpec(a.shape, lambda i: tuple(0 for _ in a.shape))
    out_shapes = (
        jax.ShapeDtypeStruct((_N, _C), _F32),
        jax.ShapeDtypeStruct((_N, _C), _BF16),
        jax.ShapeDtypeStruct((_N, _C), _BF16),
        jax.ShapeDtypeStruct((_N, 1), jnp.int32),
        jax.ShapeDtypeStruct((_N, 1), _F32),
        jax.ShapeDtypeStruct((1, 1), _F32),
    )
    out_specs = (
        row_spec,
        row_spec,
        row_spec,
        pl.BlockSpec((_BLK1, 1), lambda i: (i, 0)),
        pl.BlockSpec((_BLK1, 1), lambda i: (i, 0)),
        pl.BlockSpec((1, 1), lambda i: (0, 0), memory_space=pltpu.SMEM),
    )
    ws = [Wr, Wk, Wv, Wo, Ws, cwT, Wa, Wd8, Wb1, Wb2, l1g, l1b, l2g, l2b, lcap]
    return pl.pallas_call(
        _k1_body,
        grid=(_NB1,),
        in_specs=[row_spec, row_spec] + [full(w) for w in ws],
        out_specs=out_specs,
        out_shape=out_shapes,
        compiler_params=pltpu.CompilerParams(
            dimension_semantics=("arbitrary",)),
    )(x2d, vf2d, *ws)


_R2, _C2 = 32, 128


def _k2_body(w_ref, pos_ref, offs_ref):
    w = w_ref[...]
    iu = lax.broadcasted_iota(jnp.int32, (_C2, _C2), 0)
    ju = lax.broadcasted_iota(jnp.int32, (_C2, _C2), 1)
    U = (iu < ju).astype(_BF16)
    il = lax.broadcasted_iota(jnp.int32, (_R2, _R2), 0)
    jl = lax.broadcasted_iota(jnp.int32, (_R2, _R2), 1)
    L = (jl < il).astype(_BF16)
    pos = jnp.zeros((_R2, _C2), _F32)
    off = jnp.float32(0.0)
    for e in range(_E):
        m = (w == e).astype(_F32)
        pref = _bdot(m.astype(_BF16), U)
        tot = jnp.sum(m, axis=1, keepdims=True)
        rowpref = _bdot(L, tot.astype(_BF16))
        offs_ref[e, 0] = off.astype(jnp.int32)
        pos = pos + m * (off + rowpref + pref)
        off = off + jnp.sum(m)
    for e in range(_E, 16):
        offs_ref[e, 0] = jnp.int32(_N)
    pos_ref[...] = pos.astype(jnp.int32)


def _k2(win2d32):
    return pl.pallas_call(
        _k2_body,
        grid=(1,),
        in_specs=[pl.BlockSpec((_R2, _C2), lambda i: (0, 0))],
        out_specs=(
            pl.BlockSpec((_R2, _C2), lambda i: (0, 0)),
            pl.BlockSpec((16, 1), lambda i: (0, 0), memory_space=pltpu.SMEM),
        ),
        out_shape=(
            jax.ShapeDtypeStruct((_R2, _C2), jnp.int32),
            jax.ShapeDtypeStruct((16, 1), jnp.int32),
        ),
    )(win2d32)


_M4 = 256
_NB4 = _N // _M4
_G4 = _NB4 + _E - 1


def _k4_body(bm_r, exc_r, wmx_r, first_r, exraw_r, offs_r,
             hs_r, ss_r, x1_r, w1_r, w2_r, wm_r, out_r, acc_r):
    g = pl.program_id(0)
    e = exraw_r[g]
    lo = offs_r[e]
    hi = offs_r[e + 1]
    istrans = e >= _NUM_RWKV

    @pl.when(istrans)
    def _():
        acc_r[...] = hs_r[...].astype(_F32) + _ddot(ss_r[...].astype(_F32),
                                                    wm_r[0])

    @pl.when(jnp.logical_not(istrans))
    def _():
        acc_r[...] = hs_r[...].astype(_F32)

    u = jax.nn.relu(_ddot(acc_r[...], w1_r[0]))
    eo = _ddot(u, w2_r[0])
    rows = bm_r[g] * _M4 + lax.broadcasted_iota(jnp.int32, (_M4, 1), 0)
    inb = jnp.logical_and(rows >= lo, rows < hi)
    contrib = jnp.where(inb, eo, 0.0)
    isfirst = first_r[g] == 1

    @pl.when(isfirst)
    def _():
        out_r[...] = x1_r[...] + contrib

    @pl.when(jnp.logical_not(isfirst))
    def _():
        out_r[...] += contrib


def _k4(bm, exc, wmx, first, exraw, offs, hs_s, ss_s, x1_s, W1b, W2b, Wmb):
    row_spec = pl.BlockSpec((_M4, _C), lambda g, *s: (s[0][g], 0))
    grid_spec = pltpu.PrefetchScalarGridSpec(
        num_scalar_prefetch=6,
        grid=(_G4,),
        in_specs=[
            row_spec, row_spec, row_spec,
            pl.BlockSpec((1, _C, _C), lambda g, *s: (s[1][g], 0, 0)),
            pl.BlockSpec((1, _C, _C), lambda g, *s: (s[1][g], 0, 0)),
            pl.BlockSpec((1, _C, _C), lambda g, *s: (s[2][g], 0, 0)),
        ],
        out_specs=pl.BlockSpec((_M4, _C), lambda g, *s: (s[0][g], 0)),
        scratch_shapes=[pltpu.VMEM((_M4, _C), _F32)],
    )
    return pl.pallas_call(
        _k4_body,
        grid_spec=grid_spec,
        out_shape=jax.ShapeDtypeStruct((_N, _C), _F32),
        compiler_params=pltpu.CompilerParams(
            dimension_semantics=("arbitrary",)),
    )(bm, exc, wmx, first, exraw, offs, hs_s, ss_s, x1_s, W1b, W2b, Wmb)


def _schedule(offs):
    starts = offs[0:_E]
    ends = offs[1:_E + 1]
    b = jnp.arange(_NB4, dtype=jnp.int32)[:, None]
    inc = jnp.logical_and(starts[None, :] < (b + 1) * _M4,
                          ends[None, :] > b * _M4)
    tot = jnp.cumsum(inc.reshape(-1).astype(jnp.int32))
    kg = jnp.searchsorted(tot, jnp.arange(1, _G4 + 1, dtype=jnp.int32),
                          side='left').astype(jnp.int32)
    valid = jnp.arange(_G4, dtype=jnp.int32) < tot[-1]
    bm = jnp.where(valid, kg // _E, _NB4 - 1).astype(jnp.int32)
    ex = jnp.where(valid, kg % _E, _E).astype(jnp.int32)
    exc = jnp.minimum(ex, _E - 1)
    wmx = jnp.clip(ex - _NUM_RWKV, 0, 1)
    first = jnp.concatenate(
        [jnp.ones((1,), jnp.int32), (bm[1:] != bm[:-1]).astype(jnp.int32)])
    return bm, exc, wmx, first, ex


def kernel(x, v_first, capital_shares, step, warmup_steps, ln1_g, ln1_b, ln2_g, ln2_b, Wr, Wk, Wv, Wo, Ws, conf_W, W1, W2, Wmix, Wd, Wa, Wb1, Wb2):
    C = _C
    x2d = x.reshape(_N, C)
    vf2d = v_first.reshape(_N, C)
    Wd8 = jnp.pad(Wd, ((0, 0), (0, 7)))
    lcap = jnp.log(capital_shares + 1e-6)[None, :]
    x1, hs, ss, win2d, cost2d, sse = _k1(
        x2d, vf2d, Wr, Wk, Wv, Wo, Ws,
        conf_W.T, Wa, Wd8, Wb1, Wb2,
        ln1_g[None, :], ln1_b[None, :], ln2_g[None, :], ln2_b[None, :], lcap)
    winners = win2d[:, 0]
    costs = cost2d[:, 0]
    recon_loss = sse[0, 0] / (_N * C)

    pos2d, offs16 = _k2(win2d.reshape(_R2, _C2))
    pos = pos2d.reshape(_N)
    offs = offs16[:, 0]
    bm, exc, wmx, first, exraw = _schedule(offs)
    order = jnp.zeros((_N,), jnp.int32).at[pos].set(
        jnp.arange(_N, dtype=jnp.int32))
    hs_s = jnp.take(hs, order, axis=0)
    ss_s = jnp.take(ss, order, axis=0)
    x1_s = jnp.take(x1, order, axis=0)
    out_sorted = _k4(bm, exc, wmx, first, exraw, offs,
                     hs_s, ss_s, x1_s, W1, W2, Wmix)
    out = jnp.take(out_sorted, pos, axis=0).reshape(_B, _T, C)
    return (out, v_first, winners.reshape(_B, _T), costs.reshape(_B, _T),
            recon_loss)


# speedup vs baseline: 1.8850x; 1.3777x over previous
import functools

import jax
import jax.numpy as jnp
from jax import lax
from jax.experimental import pallas as pl
from jax.experimental.pallas import tpu as pltpu
from jax.experimental.pallas import tpu_sc as plsc

_B, _T, _C = 2, 2048, 768
_NUM_RWKV, _NUM_TRANS = 6, 2
_E = _NUM_RWKV + _NUM_TRANS
_N = _B * _T

_BLK1 = 512
_NB1 = _N // _BLK1

_F32 = jnp.float32
_BF16 = jnp.bfloat16


def _bdot(a, b):
    return lax.dot_general(a, b, (((1,), (0,)), ((), ())),
                           preferred_element_type=_F32)


def _ddot(a, b):
    return lax.dot_general(a, b, (((1,), (0,)), ((), ())),
                           precision=lax.Precision.DEFAULT,
                           preferred_element_type=_F32)


def _ln_rows(x, g, b):
    mu = jnp.mean(x, axis=1, keepdims=True)
    d = x - mu
    var = jnp.mean(d * d, axis=1, keepdims=True)
    return d / jnp.sqrt(var + 1e-5) * g + b


def _k1_body(x_ref, vf_ref, wr_ref, wk_ref, wv_ref, wo_ref, ws_ref,
             cw_ref, wa_ref, wd_ref, wb1_ref, wb2_ref,
             l1g_ref, l1b_ref, l2g_ref, l2b_ref, lcap_ref,
             x1_ref, hs_ref, ss_ref, win_ref, cost_ref, sse_ref):
    i = pl.program_id(0)
    xf = x_ref[...]
    xn = _ln_rows(xf, l1g_ref[...], l1b_ref[...])
    r = _ddot(xn, wr_ref[...])
    k = _ddot(xn, wk_ref[...])
    v0 = _ddot(xn, wv_ref[...])
    v = v0 + (vf_ref[...] - v0) * jax.nn.sigmoid(k)
    att = _ddot(jax.nn.sigmoid(r) * v, wo_ref[...])
    state = jnp.tanh(_ddot(xn, ws_ref[...]))
    x1 = xf + att
    x1_ref[...] = x1
    h = _ln_rows(x1, l2g_ref[...], l2b_ref[...])
    conf = jax.nn.sigmoid(_ddot(h, cw_ref[...]))
    aff = _ddot(h, wa_ref[...])
    dcol = _ddot(h, wd_ref[...])[:, 0:1]
    diff = jax.nn.softplus(dcol)
    eff = conf * diff + 0.1 * aff + lcap_ref[...]
    costs = jnp.max(eff, axis=1)
    ids = lax.broadcasted_iota(jnp.int32, (_BLK1, _E), 1)
    win = jnp.min(jnp.where(eff == costs[:, None], ids, _E), axis=1)
    win_ref[...] = win[:, None]
    cost_ref[...] = costs[:, None]
    wc = jnp.sum(jnp.where(ids == win[:, None], conf, 0.0), axis=1)
    scale = (wc / (wc + 1e-6))[:, None]
    hs_ref[...] = h * scale
    ss_ref[...] = state * scale
    t1 = jnp.tanh(_ddot(state, wb1_ref[...]))
    recon = _ddot(t1, wb2_ref[...])
    dsse = jnp.sum((recon - h) ** 2)

    @pl.when(i == 0)
    def _():
        sse_ref[0, 0] = dsse

    @pl.when(i != 0)
    def _():
        sse_ref[0, 0] += dsse


def _k1(x2d, vf2d, Wr, Wk, Wv, Wo, Ws, cwT, Wa, Wd8, Wb1, Wb2,
        l1g, l1b, l2g, l2b, lcap):
    row_spec = pl.BlockSpec((_BLK1, _C), lambda i: (i, 0))
    full = lambda a: pl.BlockSpec(a.shape, lambda i: tuple(0 for _ in a.shape))
    out_shapes = (
        jax.ShapeDtypeStruct((_N, _C), _F32),
        jax.ShapeDtypeStruct((_N, _C), _F32),
        jax.ShapeDtypeStruct((_N, _C), _F32),
        jax.ShapeDtypeStruct((_N, 1), jnp.int32),
        jax.ShapeDtypeStruct((_N, 1), _F32),
        jax.ShapeDtypeStruct((1, 1), _F32),
    )
    out_specs = (
        row_spec,
        row_spec,
        row_spec,
        pl.BlockSpec((_BLK1, 1), lambda i: (i, 0)),
        pl.BlockSpec((_BLK1, 1), lambda i: (i, 0)),
        pl.BlockSpec((1, 1), lambda i: (0, 0), memory_space=pltpu.SMEM),
    )
    ws = [Wr, Wk, Wv, Wo, Ws, cwT, Wa, Wd8, Wb1, Wb2, l1g, l1b, l2g, l2b, lcap]
    return pl.pallas_call(
        _k1_body,
        grid=(_NB1,),
        in_specs=[row_spec, row_spec] + [full(w) for w in ws],
        out_specs=out_specs,
        out_shape=out_shapes,
        compiler_params=pltpu.CompilerParams(
            dimension_semantics=("arbitrary",)),
    )(x2d, vf2d, *ws)


_R2, _C2 = 32, 128


def _k2_body(w_ref, pos_ref, offs_ref):
    w = w_ref[...]
    iu = lax.broadcasted_iota(jnp.int32, (_C2, _C2), 0)
    ju = lax.broadcasted_iota(jnp.int32, (_C2, _C2), 1)
    U = (iu < ju).astype(_BF16)
    il = lax.broadcasted_iota(jnp.int32, (_R2, _R2), 0)
    jl = lax.broadcasted_iota(jnp.int32, (_R2, _R2), 1)
    L = (jl < il).astype(_BF16)
    pos = jnp.zeros((_R2, _C2), _F32)
    off = jnp.float32(0.0)
    for e in range(_E):
        m = (w == e).astype(_F32)
        pref = _bdot(m.astype(_BF16), U)
        tot = jnp.sum(m, axis=1, keepdims=True)
        rowpref = _bdot(L, tot.astype(_BF16))
        offs_ref[e, 0] = off.astype(jnp.int32)
        pos = pos + m * (off + rowpref + pref)
        off = off + jnp.sum(m)
    for e in range(_E, 16):
        offs_ref[e, 0] = jnp.int32(_N)
    pos_ref[...] = pos.astype(jnp.int32)


def _k2(win2d32):
    return pl.pallas_call(
        _k2_body,
        grid=(1,),
        in_specs=[pl.BlockSpec((_R2, _C2), lambda i: (0, 0))],
        out_specs=(
            pl.BlockSpec((_R2, _C2), lambda i: (0, 0)),
            pl.BlockSpec((16, 1), lambda i: (0, 0), memory_space=pltpu.SMEM),
        ),
        out_shape=(
            jax.ShapeDtypeStruct((_R2, _C2), jnp.int32),
            jax.ShapeDtypeStruct((16, 1), jnp.int32),
        ),
    )(win2d32)


_M4 = 256
_NB4 = _N // _M4
_G4 = _NB4 + _E - 1


def _k4_body(bm_r, exc_r, wmx_r, first_r, exraw_r, offs_r,
             hs_r, ss_r, x1_r, w1_r, w2_r, wm_r, out_r, acc_r):
    g = pl.program_id(0)
    e = exraw_r[g]
    lo = offs_r[e]
    hi = offs_r[e + 1]
    istrans = e >= _NUM_RWKV

    @pl.when(istrans)
    def _():
        acc_r[...] = hs_r[...] + _ddot(ss_r[...], wm_r[0])

    @pl.when(jnp.logical_not(istrans))
    def _():
        acc_r[...] = hs_r[...]

    u = jax.nn.relu(_ddot(acc_r[...], w1_r[0]))
    eo = _ddot(u, w2_r[0])
    rows = bm_r[g] * _M4 + lax.broadcasted_iota(jnp.int32, (_M4, 1), 0)
    inb = jnp.logical_and(rows >= lo, rows < hi)
    contrib = jnp.where(inb, eo, 0.0)
    isfirst = first_r[g] == 1

    @pl.when(isfirst)
    def _():
        out_r[...] = x1_r[...] + contrib

    @pl.when(jnp.logical_not(isfirst))
    def _():
        out_r[...] += contrib


def _k4(bm, exc, wmx, first, exraw, offs, hs_s, ss_s, x1_s, W1b, W2b, Wmb):
    row_spec = pl.BlockSpec((_M4, _C), lambda g, *s: (s[0][g], 0))
    grid_spec = pltpu.PrefetchScalarGridSpec(
        num_scalar_prefetch=6,
        grid=(_G4,),
        in_specs=[
            row_spec, row_spec, row_spec,
            pl.BlockSpec((1, _C, _C), lambda g, *s: (s[1][g], 0, 0)),
            pl.BlockSpec((1, _C, _C), lambda g, *s: (s[1][g], 0, 0)),
            pl.BlockSpec((1, _C, _C), lambda g, *s: (s[2][g], 0, 0)),
        ],
        out_specs=pl.BlockSpec((_M4, _C), lambda g, *s: (s[0][g], 0)),
        scratch_shapes=[pltpu.VMEM((_M4, _C), _F32)],
    )
    return pl.pallas_call(
        _k4_body,
        grid_spec=grid_spec,
        out_shape=jax.ShapeDtypeStruct((_N, _C), _F32),
        compiler_params=pltpu.CompilerParams(
            dimension_semantics=("arbitrary",)),
    )(bm, exc, wmx, first, exraw, offs, hs_s, ss_s, x1_s, W1b, W2b, Wmb)


def _schedule(offs):
    starts = offs[0:_E]
    ends = offs[1:_E + 1]
    b = jnp.arange(_NB4, dtype=jnp.int32)[:, None]
    inc = jnp.logical_and(starts[None, :] < (b + 1) * _M4,
                          ends[None, :] > b * _M4)
    tot = jnp.cumsum(inc.reshape(-1).astype(jnp.int32))
    kg = jnp.searchsorted(tot, jnp.arange(1, _G4 + 1, dtype=jnp.int32),
                          side='left').astype(jnp.int32)
    valid = jnp.arange(_G4, dtype=jnp.int32) < tot[-1]
    bm = jnp.where(valid, kg // _E, _NB4 - 1).astype(jnp.int32)
    ex = jnp.where(valid, kg % _E, _E).astype(jnp.int32)
    exc = jnp.minimum(ex, _E - 1)
    wmx = jnp.clip(ex - _NUM_RWKV, 0, 1)
    first = jnp.concatenate(
        [jnp.ones((1,), jnp.int32), (bm[1:] != bm[:-1]).astype(jnp.int32)])
    return bm, exc, wmx, first, ex


_NC, _NS = 2, 16
_NW = _NC * _NS
_CH = _N // _NW
_HC = _CH // 2


def _k3_body(pos_r, hs_r, ss_r, x1_r, hs_o, ss_o, x1_o,
             idx_v, buf0, buf1, si0, si1, so0, so1):
    wid = lax.axis_index("s") * _NC + lax.axis_index("c")
    base = wid * _CH
    pltpu.sync_copy(pos_r.at[wid], idx_v)
    srcs = (hs_r, ss_r, x1_r)
    dsts = (hs_o, ss_o, x1_o)
    bufs = (buf0, buf1)
    sin = (si0, si1)
    sout = (so0, so1)

    def in_copy(t):
        return pltpu.make_async_copy(
            srcs[t // 2].at[pl.ds(base + (t % 2) * _HC, _HC)],
            bufs[t % 2], sin[t % 2])

    def out_copy(t):
        return pltpu.make_async_copy(
            bufs[t % 2], dsts[t // 2].at[idx_v.at[t % 2]], sout[t % 2])

    in_copy(0).start()
    for t in range(6):
        in_copy(t).wait()
        if t >= 1:
            out_copy(t - 1).wait()
        if t + 1 < 6:
            in_copy(t + 1).start()
        out_copy(t).start()
    out_copy(5).wait()


def _k3(pos3d, hs, ss, x1):
    mesh = plsc.VectorSubcoreMesh(core_axis_name="c", subcore_axis_name="s")
    sds = jax.ShapeDtypeStruct((_N, _C), _F32)
    run = functools.partial(
        pl.kernel, mesh=mesh,
        out_type=(sds, sds, sds),
        scratch_types=[
            pltpu.VMEM((2, _HC), jnp.int32),
            pltpu.VMEM((_HC, _C), _F32),
            pltpu.VMEM((_HC, _C), _F32),
            pltpu.SemaphoreType.DMA,
            pltpu.SemaphoreType.DMA,
            pltpu.SemaphoreType.DMA,
            pltpu.SemaphoreType.DMA,
        ])(_k3_body)
    return run(pos3d, hs, ss, x1)


def _k5_body(pos_r, os_r, xo_r, idx_v, buf, sem):
    wid = lax.axis_index("s") * _NC + lax.axis_index("c")
    base = wid * _CH
    pltpu.sync_copy(pos_r.at[wid], idx_v)
    pltpu.async_copy(os_r.at[idx_v], buf, sem).wait()
    pltpu.sync_copy(buf, xo_r.at[pl.ds(base, _CH)])


def _k5(pos2d, out_sorted):
    mesh = plsc.VectorSubcoreMesh(core_axis_name="c", subcore_axis_name="s")
    run = functools.partial(
        pl.kernel, mesh=mesh,
        out_type=jax.ShapeDtypeStruct((_N, _C), _F32),
        scratch_types=[
            pltpu.VMEM((_CH,), jnp.int32),
            pltpu.VMEM((_CH, _C), _F32),
            pltpu.SemaphoreType.DMA,
        ])(_k5_body)
    return run(pos2d, out_sorted)


def kernel(x, v_first, capital_shares, step, warmup_steps, ln1_g, ln1_b, ln2_g, ln2_b, Wr, Wk, Wv, Wo, Ws, conf_W, W1, W2, Wmix, Wd, Wa, Wb1, Wb2):
    C = _C
    x2d = x.reshape(_N, C)
    vf2d = v_first.reshape(_N, C)
    Wd8 = jnp.pad(Wd, ((0, 0), (0, 7)))
    lcap = jnp.log(capital_shares + 1e-6)[None, :]
    x1, hs, ss, win2d, cost2d, sse = _k1(
        x2d, vf2d, Wr, Wk, Wv, Wo, Ws,
        conf_W.T, Wa, Wd8, Wb1, Wb2,
        ln1_g[None, :], ln1_b[None, :], ln2_g[None, :], ln2_b[None, :], lcap)
    winners = win2d[:, 0]
    costs = cost2d[:, 0]
    recon_loss = sse[0, 0] / (_N * C)

    pos2d, offs16 = _k2(win2d.reshape(_R2, _C2))
    pos = pos2d.reshape(_N)
    offs = offs16[:, 0]
    bm, exc, wmx, first, exraw = _schedule(offs)
    hs_s, ss_s, x1_s = _k3(pos2d.reshape(_NW, 2, _HC), hs, ss, x1)
    out_sorted = _k4(bm, exc, wmx, first, exraw, offs,
                     hs_s, ss_s, x1_s, W1, W2, Wmix)
    out = _k5(pos2d.reshape(_NW, _CH), out_sorted).reshape(_B, _T, C)
    return (out, v_first, winners.reshape(_B, _T), costs.reshape(_B, _T),
            recon_loss)
